# baseline (speedup 1.0000x reference)
import jax
import jax.numpy as jnp
from jax import lax
from jax.experimental import pallas as pl
from jax.experimental.pallas import tpu as pltpu
from jax.experimental.pallas import tpu_sc as plsc

_B, _T, _C = 16, 512, 128
_BLANK = _C - 1
_NBLK = _T // 16
_LN2 = 0.6931471805599453
_SQRT2 = 1.4142135623730951


def _splat_i32(v):
    return jnp.full((16,), v, jnp.int32)


def _ln(s):
    bits = lax.bitcast_convert_type(s, jnp.int32)
    e = (lax.shift_right_logical(bits, 23) & 0xFF) - 127
    f = lax.bitcast_convert_type((bits & 0x007FFFFF) | 0x3F800000, jnp.float32)
    big = f > _SQRT2
    f = jnp.where(big, f * 0.5, f)
    e = jnp.where(big, e + 1, e)
    u = (f - 1.0) / (f + 1.0)
    u2 = u * u
    poly = 2.0 * u * (1.0 + u2 * (1.0 / 3.0 + u2 * (
        0.2 + u2 * (1.0 / 7.0 + u2 * (1.0 / 9.0)))))
    return e.astype(jnp.float32) * _LN2 + poly


def _sc_body(x_hbm, dec_hbm, prob_hbm,
             xbuf, negl, lp2b, tok1b, tok2b, tokc, decb, tmp, pbuf):
    core = lax.axis_index("c")
    tile = lax.axis_index("s")
    lane = lax.iota(jnp.int32, 16)

    @pl.when(core == 0)
    def _work():
        b = tile
        pltpu.sync_copy(x_hbm.at[b], xbuf)

        def blk_a(i, carry):
            rows = (i * 16 + lane) * _C

            def cstep(ci, st):
                base = ci + lane
                new = []
                for j in range(4):
                    v1, i1, v2, i2, acc = st[j]
                    civ = (base + 32 * j) & 127
                    v = plsc.load_gather(xbuf, [rows + civ])
                    gt1 = v > v1
                    gt2 = v > v2
                    v2n = jnp.where(gt1, v1, jnp.where(gt2, v, v2))
                    i2n = jnp.where(gt1, i1, jnp.where(gt2, civ, i2))
                    v1n = jnp.where(gt1, v, v1)
                    i1n = jnp.where(gt1, civ, i1)
                    new.append((v1n, i1n, v2n, i2n, acc + jnp.exp(v)))
                return tuple(new)

            ninf = jnp.full((16,), -jnp.inf, jnp.float32)
            z = jnp.zeros((16,), jnp.float32)
            st0 = ((ninf, _splat_i32(0), ninf, _splat_i32(0), z),) * 4
            st = lax.fori_loop(0, 32, cstep, st0, unroll=32)
            parts = [s[:4] for s in st]

            def merge(a, b):
                av1, ai1, av2, ai2 = a
                bv1, bi1, bv2, bi2 = b
                aw = av1 >= bv1
                v1 = jnp.where(aw, av1, bv1)
                i1 = jnp.where(aw, ai1, bi1)
                c2v = jnp.where(aw, av2, av1)
                c2i = jnp.where(aw, ai2, ai1)
                d2v = jnp.where(aw, bv1, bv2)
                d2i = jnp.where(aw, bi1, bi2)
                s2 = c2v >= d2v
                return (v1, i1, jnp.where(s2, c2v, d2v),
                        jnp.where(s2, c2i, d2i))

            v1, i1, v2, i2 = merge(merge(parts[0], parts[1]),
                                   merge(parts[2], parts[3]))
            ssum = (st[0][4] + st[1][4]) + (st[2][4] + st[3][4])
            d = _ln(ssum)
            nl = v1 - d
            negl[pl.ds(i * 16, 16)] = nl
            lp2b[pl.ds(i * 16, 16)] = (v2 - v1) + nl
            tok1b[pl.ds(i * 16, 16)] = i1
            tok2b[pl.ds(i * 16, 16)] = i2
            return carry

        lax.fori_loop(0, _NBLK, blk_a, 0)

        tokc[pl.ds(0, 16)] = _splat_i32(-1)

        def blk_b(i, carry_s):
            nl = negl[pl.ds(i * 16, 16)]
            incl = plsc.cumsum(nl)
            ex = carry_s + (incl - nl)
            a = ex + nl
            bb = ex + lp2b[pl.ds(i * 16, 16)]
            t1 = tok1b[pl.ds(i * 16, 16)]
            t2 = tok2b[pl.ds(i * 16, 16)]
            tie = (bb == a) & (t2 < t1)
            tok = jnp.where(tie, t2, t1)
            plsc.store_scatter(tokc, [i * 16 + 1 + lane], tok)
            prv = plsc.load_gather(tokc, [i * 16 + lane])
            keep = (tok != prv) & (tok != _BLANK)
            decb[pl.ds(i * 16, 16)] = jnp.where(keep, tok, _splat_i32(-1))
            tmp[...] = a
            return plsc.load_gather(tmp, [_splat_i32(15)])

        s_vec = lax.fori_loop(0, _NBLK, blk_b,
                              jnp.zeros((16,), jnp.float32))
        pltpu.sync_copy(decb, dec_hbm.at[b])

        pbuf[...] = jnp.exp(s_vec)
        pltpu.sync_copy(pbuf, prob_hbm.at[b])


def kernel(inputs):
    mesh = plsc.VectorSubcoreMesh(core_axis_name="c", subcore_axis_name="s")
    dec, prob16 = pl.kernel(
        _sc_body,
        out_type=(
            jax.ShapeDtypeStruct((_B, _T), jnp.int32),
            jax.ShapeDtypeStruct((_B, 16), jnp.float32),
        ),
        mesh=mesh,
        compiler_params=pltpu.CompilerParams(needs_layout_passes=False),
        scratch_types=[
            pltpu.VMEM((_T * _C,), jnp.float32),
            pltpu.VMEM((_T,), jnp.float32),
            pltpu.VMEM((_T,), jnp.float32),
            pltpu.VMEM((_T,), jnp.int32),
            pltpu.VMEM((_T,), jnp.int32),
            pltpu.VMEM((_T + 16,), jnp.int32),
            pltpu.VMEM((_T,), jnp.int32),
            pltpu.VMEM((16,), jnp.float32),
            pltpu.VMEM((16,), jnp.float32),
        ],
    )(inputs.reshape(_B, _T * _C))
    return dec.reshape(_B, 1, _T), prob16[:, :1]


# speedup vs baseline: 1.0021x; 1.0021x over previous
import jax
import jax.numpy as jnp
from jax import lax
from jax.experimental import pallas as pl
from jax.experimental.pallas import tpu as pltpu
from jax.experimental.pallas import tpu_sc as plsc

_B, _T, _C = 16, 512, 128
_BLANK = _C - 1
_NBLK = _T // 16
_LN2 = 0.6931471805599453
_SQRT2 = 1.4142135623730951


def _splat_i32(v):
    return jnp.full((16,), v, jnp.int32)


def _ln(s):
    bits = lax.bitcast_convert_type(s, jnp.int32)
    e = (lax.shift_right_logical(bits, 23) & 0xFF) - 127
    f = lax.bitcast_convert_type((bits & 0x007FFFFF) | 0x3F800000, jnp.float32)
    big = f > _SQRT2
    f = jnp.where(big, f * 0.5, f)
    e = jnp.where(big, e + 1, e)
    u = (f - 1.0) / (f + 1.0)
    u2 = u * u
    poly = 2.0 * u * (1.0 + u2 * (1.0 / 3.0 + u2 * (
        0.2 + u2 * (1.0 / 7.0 + u2 * (1.0 / 9.0)))))
    return e.astype(jnp.float32) * _LN2 + poly


def _sc_body(x_hbm, dec_hbm, prob_hbm,
             xbuf, negl, lp2b, tok1b, tok2b, tokc, decb, tmp, pbuf):
    core = lax.axis_index("c")
    tile = lax.axis_index("s")
    lane = lax.iota(jnp.int32, 16)

    @pl.when(core == 0)
    def _work():
        b = tile
        pltpu.sync_copy(x_hbm.at[b], xbuf)

        def blk_a(i, carry):
            rows = (i * 16 + lane) * _C

            def cstep(ci, st):
                base = ci + lane
                new = []
                for j in range(4):
                    v1, i1, v2, i2, acc = st[j]
                    civ = (base + 32 * j) & 127
                    v = plsc.load_gather(xbuf, [rows + civ])
                    gt1 = v > v1
                    gt2 = v > v2
                    v2n = jnp.where(gt1, v1, jnp.where(gt2, v, v2))
                    i2n = jnp.where(gt1, i1, jnp.where(gt2, civ, i2))
                    v1n = jnp.where(gt1, v, v1)
                    i1n = jnp.where(gt1, civ, i1)
                    new.append((v1n, i1n, v2n, i2n, acc + jnp.exp(v)))
                return tuple(new)

            ninf = jnp.full((16,), -jnp.inf, jnp.float32)
            z = jnp.zeros((16,), jnp.float32)
            st0 = ((ninf, _splat_i32(0), ninf, _splat_i32(0), z),) * 4
            st = lax.fori_loop(0, 32, cstep, st0, unroll=32)
            parts = [s[:4] for s in st]

            def merge(a, b):
                av1, ai1, av2, ai2 = a
                bv1, bi1, bv2, bi2 = b
                aw = av1 >= bv1
                v1 = jnp.where(aw, av1, bv1)
                i1 = jnp.where(aw, ai1, bi1)
                c2v = jnp.where(aw, av2, av1)
                c2i = jnp.where(aw, ai2, ai1)
                d2v = jnp.where(aw, bv1, bv2)
                d2i = jnp.where(aw, bi1, bi2)
                s2 = c2v >= d2v
                return (v1, i1, jnp.where(s2, c2v, d2v),
                        jnp.where(s2, c2i, d2i))

            v1, i1, v2, i2 = merge(merge(parts[0], parts[1]),
                                   merge(parts[2], parts[3]))
            ssum = (st[0][4] + st[1][4]) + (st[2][4] + st[3][4])
            d = _ln(ssum)
            nl = v1 - d
            negl[pl.ds(i * 16, 16)] = nl
            lp2b[pl.ds(i * 16, 16)] = (v2 - v1) + nl
            tok1b[pl.ds(i * 16, 16)] = i1
            tok2b[pl.ds(i * 16, 16)] = i2
            return carry

        lax.fori_loop(0, _NBLK, blk_a, 0)

        tokc[pl.ds(0, 16)] = _splat_i32(-1)

        def blk_b(i, carry_s):
            nl = negl[pl.ds(i * 16, 16)]
            incl = plsc.cumsum(nl)
            ex = carry_s + (incl - nl)
            a = ex + nl
            bb = ex + lp2b[pl.ds(i * 16, 16)]
            t1 = tok1b[pl.ds(i * 16, 16)]
            t2 = tok2b[pl.ds(i * 16, 16)]
            tie = (bb == a) & (t2 < t1)
            tok = jnp.where(tie, t2, t1)
            plsc.store_scatter(tokc, [i * 16 + 1 + lane], tok)
            prv = plsc.load_gather(tokc, [i * 16 + lane])
            keep = (tok != prv) & (tok != _BLANK)
            decb[pl.ds(i * 16, 16)] = jnp.where(keep, tok, _splat_i32(-1))
            return carry_s + jnp.full((16,), jnp.sum(nl), jnp.float32)

        s_vec = lax.fori_loop(0, _NBLK, blk_b,
                              jnp.zeros((16,), jnp.float32))
        pltpu.sync_copy(decb, dec_hbm.at[b])

        pbuf[...] = jnp.exp(s_vec)
        pltpu.sync_copy(pbuf, prob_hbm.at[b])


def kernel(inputs):
    mesh = plsc.VectorSubcoreMesh(core_axis_name="c", subcore_axis_name="s")
    dec, prob16 = pl.kernel(
        _sc_body,
        out_type=(
            jax.ShapeDtypeStruct((_B, _T), jnp.int32),
            jax.ShapeDtypeStruct((_B, 16), jnp.float32),
        ),
        mesh=mesh,
        compiler_params=pltpu.CompilerParams(needs_layout_passes=False),
        scratch_types=[
            pltpu.VMEM((_T * _C,), jnp.float32),
            pltpu.VMEM((_T,), jnp.float32),
            pltpu.VMEM((_T,), jnp.float32),
            pltpu.VMEM((_T,), jnp.int32),
            pltpu.VMEM((_T,), jnp.int32),
            pltpu.VMEM((_T + 16,), jnp.int32),
            pltpu.VMEM((_T,), jnp.int32),
            pltpu.VMEM((16,), jnp.float32),
            pltpu.VMEM((16,), jnp.float32),
        ],
    )(inputs.reshape(_B, _T * _C))
    return dec.reshape(_B, 1, _T), prob16[:, :1]
